# Initial kernel scaffold; baseline (speedup 1.0000x reference)
import functools

import jax
import jax.numpy as jnp
from jax import lax
from jax.experimental import pallas as pl
from jax.experimental.pallas import tpu as pltpu
from jax.experimental.pallas import tpu_sc as plsc

_f32 = jnp.float32
_i32 = jnp.int32

NC = 2
NS = 16
NW = NC * NS
LANES = 16
CH = 80
ZCH = 125


def _mesh():
    return plsc.VectorSubcoreMesh(core_axis_name="c", subcore_axis_name="s")


def _sc_degree(dst_r, n):
    nw, nch, c = dst_r.shape
    rpt = n // NS
    nz = rpt // ZCH

    @functools.partial(
        pl.kernel,
        out_type=jax.ShapeDtypeStruct((NC, n, 16), _f32),
        mesh=_mesh(),
        scratch_types=[
            pltpu.VMEM((nch, c), _i32),
            pltpu.VMEM((c, 16), _f32),
            pltpu.VMEM((ZCH, 16), _f32),
            pltpu.VMEM_SHARED((n, 16), _f32),
        ],
    )
    def k(dst_hbm, out_hbm, didx_v, ones_v, zb_v, acc):
        cid = lax.axis_index("c")
        sid = lax.axis_index("s")
        w = cid * NS + sid
        pltpu.sync_copy(dst_hbm.at[w], didx_v)

        def fill(i, carry):
            zb_v[i, :] = jnp.zeros((LANES,), _f32)
            return carry

        lax.fori_loop(0, ZCH, fill, 0)

        def fillo(i, carry):
            ones_v[i, :] = jnp.full((LANES,), 1.0, _f32)
            return carry

        lax.fori_loop(0, c, fillo, 0)
        base = sid * rpt
        for z in range(nz):
            pltpu.sync_copy(zb_v, acc.at[pl.ds(base + z * ZCH, ZCH)])
        plsc.subcore_barrier()

        def body(j, carry):
            pltpu.sync_copy(ones_v, acc.at[didx_v.at[j]], add=True)
            return carry

        lax.fori_loop(0, nch, body, 0)
        plsc.subcore_barrier()
        for z in range(nz):
            sl = pl.ds(base + z * ZCH, ZCH)
            pltpu.sync_copy(acc.at[sl], out_hbm.at[cid, sl])

    return k(dst_r)


def _sc_scatter(a, src_r, dst_r, n, d):
    nw, nch, c = src_r.shape
    rpt = n // NS
    nz = rpt // ZCH

    @functools.partial(
        pl.kernel,
        out_type=jax.ShapeDtypeStruct((NC, n, d), _f32),
        mesh=_mesh(),
        scratch_types=[
            pltpu.VMEM((nch, c), _i32),
            pltpu.VMEM((nch, c), _i32),
            pltpu.VMEM((c, d), _f32),
            pltpu.VMEM((ZCH, d), _f32),
            pltpu.VMEM_SHARED((n, d), _f32),
            pltpu.SemaphoreType.DMA,
        ],
    )
    def k(a_hbm, src_hbm, dst_hbm, out_hbm, sidx_v, didx_v, rows_v, zb_v, acc, sem):
        cid = lax.axis_index("c")
        sid = lax.axis_index("s")
        w = cid * NS + sid
        pltpu.sync_copy(src_hbm.at[w], sidx_v)
        pltpu.sync_copy(dst_hbm.at[w], didx_v)
        nseg = d // LANES

        def fillz(i, carry):
            for t in range(nseg):
                zb_v[i, pl.ds(t * LANES, LANES)] = jnp.zeros((LANES,), _f32)
            return carry

        lax.fori_loop(0, ZCH, fillz, 0)
        base = sid * rpt
        for z in range(nz):
            pltpu.sync_copy(zb_v, acc.at[pl.ds(base + z * ZCH, ZCH)])
        plsc.subcore_barrier()

        def body(j, carry):
            pltpu.async_copy(a_hbm.at[sidx_v.at[j]], rows_v, sem).wait()
            pltpu.sync_copy(rows_v, acc.at[didx_v.at[j]], add=True)
            return carry

        lax.fori_loop(0, nch, body, 0)
        plsc.subcore_barrier()
        for z in range(nz):
            sl = pl.ds(base + z * ZCH, ZCH)
            pltpu.sync_copy(acc.at[sl], out_hbm.at[cid, sl])

    return k(a, src_r, dst_r)


def _sc_gat(h, als, ald, src_r, dst_r, n):
    d = 128
    heads = 4
    nw, nch, c = src_r.shape
    groups = c // LANES
    rpt = n // NS

    @functools.partial(
        pl.kernel,
        out_type=(
            jax.ShapeDtypeStruct((NC, n, d), _f32),
            jax.ShapeDtypeStruct((NC, n, 16), _f32),
        ),
        mesh=_mesh(),
        scratch_types=[
            pltpu.VMEM((n, heads), _f32),
            pltpu.VMEM((n, heads), _f32),
            pltpu.VMEM((nch, c), _i32),
            pltpu.VMEM((nch, c), _i32),
            pltpu.VMEM((c, d), _f32),
            pltpu.VMEM((c, 16), _f32),
            pltpu.VMEM((ZCH, 16), _f32),
            pltpu.VMEM_SHARED((n, d), _f32),
            pltpu.VMEM_SHARED((n, 16), _f32),
            pltpu.SemaphoreType.DMA,
        ],
    )
    def k(h_hbm, als_hbm, ald_hbm, src_hbm, dst_hbm, num_hbm, den_hbm,
          als_v, ald_v, sidx_v, didx_v, rows_v, sw_v, zd_v, nacc, dacc, sem):
        cid = lax.axis_index("c")
        sid = lax.axis_index("s")
        w = cid * NS + sid
        pltpu.sync_copy(als_hbm, als_v)
        pltpu.sync_copy(ald_hbm, ald_v)
        pltpu.sync_copy(src_hbm.at[w], sidx_v)
        pltpu.sync_copy(dst_hbm.at[w], didx_v)

        def fillz(i, carry):
            for t in range(d // LANES):
                rows_v[i, pl.ds(t * LANES, LANES)] = jnp.zeros((LANES,), _f32)
            return carry

        lax.fori_loop(0, c, fillz, 0)

        def fillz2(i, carry):
            zd_v[i, :] = jnp.zeros((LANES,), _f32)
            return carry

        lax.fori_loop(0, ZCH, fillz2, 0)
        base = sid * rpt
        for z in range(7):
            pltpu.sync_copy(rows_v, nacc.at[pl.ds(base + z * c, c)])
        pltpu.sync_copy(rows_v.at[pl.ds(0, rpt - 7 * c)],
                        nacc.at[pl.ds(base + 7 * c, rpt - 7 * c)])
        for z in range(rpt // ZCH):
            pltpu.sync_copy(zd_v, dacc.at[pl.ds(base + z * ZCH, ZCH)])
        plsc.subcore_barrier()

        def body(j, carry):
            pltpu.async_copy(h_hbm.at[sidx_v.at[j]], rows_v, sem).wait()
            for g in range(groups):
                s16 = sidx_v[j, pl.ds(g * LANES, LANES)]
                d16 = didx_v[j, pl.ds(g * LANES, LANES)]
                rid = g * LANES + lax.iota(_i32, LANES)
                for hh in range(heads):
                    hcol = jnp.full((LANES,), hh, _i32)
                    e = (plsc.load_gather(als_v, [s16, hcol]) +
                         plsc.load_gather(ald_v, [d16, hcol]))
                    e = jnp.where(e < 0.0, e * 0.2, e)
                    s = jnp.exp(e)
                    for r in range(4):
                        plsc.store_scatter(
                            sw_v, [rid, jnp.full((LANES,), hh + 4 * r, _i32)], s)

            def scale(i, carry2):
                for hh in range(heads):
                    sp = plsc.load_gather(
                        sw_v,
                        [jnp.full((LANES,), i, _i32), jnp.full((LANES,), hh, _i32)])
                    for t in range(2):
                        sl = pl.ds(hh * 32 + t * LANES, LANES)
                        rows_v[i, sl] = rows_v[i, sl] * sp
                return carry2

            lax.fori_loop(0, c, scale, 0)
            pltpu.sync_copy(rows_v, nacc.at[didx_v.at[j]], add=True)
            pltpu.sync_copy(sw_v, dacc.at[didx_v.at[j]], add=True)
            return carry

        lax.fori_loop(0, nch, body, 0)
        plsc.subcore_barrier()
        for z in range(rpt // ZCH):
            sl = pl.ds(base + z * ZCH, ZCH)
            pltpu.sync_copy(nacc.at[sl], num_hbm.at[cid, sl])
            pltpu.sync_copy(dacc.at[sl], den_hbm.at[cid, sl])

    return k(h, als, ald, src_r, dst_r)


def _tc1(x, w1, degp):
    n, din = x.shape
    dout = w1.shape[1]

    def body(x_ref, w_ref, degp_ref, a1_ref, dinv_ref):
        dsum = degp_ref[0] + degp_ref[1]
        deg = dsum[:, 0:1] + 1.0
        dinv = lax.rsqrt(deg)
        a1_ref[...] = jnp.dot(x_ref[...], w_ref[...],
                              preferred_element_type=_f32) * dinv
        dinv_ref[...] = dinv

    return pl.pallas_call(
        body,
        out_shape=(jax.ShapeDtypeStruct((n, dout), _f32),
                   jax.ShapeDtypeStruct((n, 1), _f32)),
    )(x, w1, degp)


def _tc_gcn_next(s, a, dinv, b, g, be, w_next):
    n, d = a.shape
    dout = w_next.shape[1]

    def body(s_ref, a_ref, dinv_ref, b_ref, g_ref, be_ref, w_ref, out_ref):
        y = (s_ref[0] + s_ref[1] + a_ref[...]) * dinv_ref[...] + b_ref[...]
        m = jnp.mean(y, axis=0, keepdims=True)
        v = jnp.mean((y - m) * (y - m), axis=0, keepdims=True)
        hn = g_ref[...] * (y - m) * lax.rsqrt(v + 1e-5) + be_ref[...]
        h = jnp.maximum(hn, 0.0)
        out_ref[...] = jnp.dot(h, w_ref[...],
                               preferred_element_type=_f32) * dinv_ref[...]

    return pl.pallas_call(
        body, out_shape=jax.ShapeDtypeStruct((n, dout), _f32),
    )(s, a, dinv, b, g, be, w_next)


def _tc4(s, a, dinv, b, g, be, wg, a_s_mat, a_d_mat):
    n, d = a.shape
    dg = wg.shape[1]
    heads = a_s_mat.shape[1]

    def body(s_ref, a_ref, dinv_ref, b_ref, g_ref, be_ref, wg_ref, as_ref,
             ad_ref, h_ref, als_ref, ald_ref):
        y = (s_ref[0] + s_ref[1] + a_ref[...]) * dinv_ref[...] + b_ref[...]
        m = jnp.mean(y, axis=0, keepdims=True)
        v = jnp.mean((y - m) * (y - m), axis=0, keepdims=True)
        hn = g_ref[...] * (y - m) * lax.rsqrt(v + 1e-5) + be_ref[...]
        h3 = jnp.maximum(hn, 0.0)
        hh = jnp.dot(h3, wg_ref[...], preferred_element_type=_f32)
        h_ref[...] = hh
        als_ref[...] = jnp.dot(hh, as_ref[...], preferred_element_type=_f32)
        ald_ref[...] = jnp.dot(hh, ad_ref[...], preferred_element_type=_f32)

    return pl.pallas_call(
        body,
        out_shape=(jax.ShapeDtypeStruct((n, dg), _f32),
                   jax.ShapeDtypeStruct((n, heads), _f32),
                   jax.ShapeDtypeStruct((n, heads), _f32)),
    )(s, a, dinv, b, g, be, wg, a_s_mat, a_d_mat)


def _tc5(nump, denp, h, als, ald, bg, rmat, pmat, wc1, bc1, wc2, bc2):
    n, d = h.shape

    def body(np_ref, dp_ref, h_ref, als_ref, ald_ref, bg_ref, r_ref, p_ref,
             wc1_ref, bc1_ref, wc2_ref, bc2_ref, out_ref):
        e = als_ref[...] + ald_ref[...]
        sself = jnp.exp(jnp.where(e < 0.0, 0.2 * e, e))
        scale = jnp.dot(sself, r_ref[...], preferred_element_type=_f32)
        num = np_ref[0] + np_ref[1] + h_ref[...] * scale
        dsum = dp_ref[0] + dp_ref[1]
        den4 = dsum[:, 0:4] + sself
        den = jnp.dot(den4, r_ref[...], preferred_element_type=_f32) + 1e-16
        outh = num / den
        h4 = jnp.maximum(
            jnp.dot(outh, p_ref[...], preferred_element_type=_f32) + bg_ref[...],
            0.0)
        gm = jnp.mean(h4, axis=0, keepdims=True)
        gc = jnp.maximum(
            jnp.dot(gm, wc1_ref[...], preferred_element_type=_f32) + bc1_ref[...],
            0.0)
        out_ref[...] = jnp.dot(gc, wc2_ref[...],
                               preferred_element_type=_f32) + bc2_ref[...]

    return pl.pallas_call(
        body, out_shape=jax.ShapeDtypeStruct((1, wc2.shape[1]), _f32),
    )(nump, denp, h, als, ald, bg, rmat, pmat, wc1, bc1, wc2, bc2)


def kernel(x, edge_index, params):
    n, din = x.shape
    e = edge_index.shape[1]
    nch = e // (NW * CH)
    assert e == NW * nch * CH, (e, NW, nch, CH)
    p = params

    src_r = edge_index[0].reshape(NW, nch, CH)
    dst_r = edge_index[1].reshape(NW, nch, CH)

    row = lambda v: v.reshape(1, -1)
    heads, hc = p['ag_s'].shape
    eye_h = jnp.eye(heads, dtype=_f32)
    a_s_mat = (eye_h[:, None, :] * p['ag_s'][:, :, None]).reshape(heads * hc, heads)
    a_d_mat = (eye_h[:, None, :] * p['ag_d'][:, :, None]).reshape(heads * hc, heads)
    rmat = (eye_h[:, :, None] * jnp.ones((1, 1, hc), _f32)).reshape(heads, heads * hc)
    pmat = (jnp.ones((heads, 1, 1), _f32) *
            jnp.eye(hc, dtype=_f32)[None] * (1.0 / heads)).reshape(heads * hc, hc)

    degp = _sc_degree(dst_r, n)
    a1, dinv = _tc1(x, p['W1'], degp)
    s1 = _sc_scatter(a1, src_r, dst_r, n, a1.shape[1])
    a2 = _tc_gcn_next(s1, a1, dinv, row(p['b1']), row(p['g1']), row(p['be1']),
                      p['W2'])
    s2 = _sc_scatter(a2, src_r, dst_r, n, a2.shape[1])
    a3 = _tc_gcn_next(s2, a2, dinv, row(p['b2']), row(p['g2']), row(p['be2']),
                      p['W3'])
    s3 = _sc_scatter(a3, src_r, dst_r, n, a3.shape[1])
    h, als, ald = _tc4(s3, a3, dinv, row(p['b3']), row(p['g3']), row(p['be3']),
                       p['Wg'], a_s_mat, a_d_mat)
    nump, denp = _sc_gat(h, als, ald, src_r, dst_r, n)
    return _tc5(nump, denp, h, als, ald, row(p['bg']), rmat, pmat,
                p['Wc1'], row(p['bc1']), p['Wc2'], row(p['bc2']))


# speedup vs baseline: 33.5809x; 33.5809x over previous
import functools

import jax
import jax.numpy as jnp
from jax import lax
from jax.experimental import pallas as pl
from jax.experimental.pallas import tpu as pltpu
from jax.experimental.pallas import tpu_sc as plsc

_f32 = jnp.float32
_i32 = jnp.int32

NC = 2
NS = 16
NW = NC * NS
LANES = 16
CH = 80


def _mesh():
    return plsc.VectorSubcoreMesh(core_axis_name="c", subcore_axis_name="s")


def _padrows(n):
    cpt = -(-n // (NS * CH))
    return cpt, cpt * CH, NS * cpt * CH


def _sc_degree(dst_r, n):
    nw, nch, c = dst_r.shape
    nz, rpt, npad = _padrows(n)

    @functools.partial(
        pl.kernel,
        out_type=jax.ShapeDtypeStruct((NC, npad, 16), _f32),
        mesh=_mesh(),
        compiler_params=pltpu.CompilerParams(use_tc_tiling_on_sc=False, needs_layout_passes=False),
        scratch_types=[
            pltpu.VMEM((nch, c), _i32),
            pltpu.VMEM((c, 16), _f32),
            pltpu.VMEM((CH, 16), _f32),
            pltpu.VMEM_SHARED((npad, 16), _f32),
        ],
    )
    def k(dst_hbm, out_hbm, didx_v, ones_v, zb_v, acc):
        cid = lax.axis_index("c")
        sid = lax.axis_index("s")
        w = cid * NS + sid
        pltpu.sync_copy(dst_hbm.at[w], didx_v)

        def fill(i, carry):
            zb_v[i, :] = jnp.zeros((LANES,), _f32)
            ones_v[i, :] = jnp.full((LANES,), 1.0, _f32)
            return carry

        lax.fori_loop(0, CH, fill, 0)
        base = sid * rpt
        for z in range(nz):
            pltpu.sync_copy(zb_v, acc.at[pl.ds(base + z * CH, CH)])
        plsc.subcore_barrier()

        def body(j, carry):
            pltpu.sync_copy(ones_v, acc.at[didx_v.at[j]], add=True)
            return carry

        lax.fori_loop(0, nch, body, 0)
        plsc.subcore_barrier()
        for z in range(nz):
            sl = pl.ds(base + z * CH, CH)
            pltpu.sync_copy(acc.at[sl], out_hbm.at[cid, sl])

    return k(dst_r)


def _sc_scatter(a, src_r, dst_r, n):
    d = a.shape[1]
    nw, nch, c = src_r.shape
    nz, rpt, npad = _padrows(n)

    @functools.partial(
        pl.kernel,
        out_type=jax.ShapeDtypeStruct((NC, npad, d), _f32),
        mesh=_mesh(),
        compiler_params=pltpu.CompilerParams(use_tc_tiling_on_sc=False, needs_layout_passes=False),
        scratch_types=[
            pltpu.VMEM((nch, c), _i32),
            pltpu.VMEM((nch, c), _i32),
            pltpu.VMEM((c, d), _f32),
            pltpu.VMEM((CH, d), _f32),
            pltpu.VMEM_SHARED((npad, d), _f32),
            pltpu.SemaphoreType.DMA,
        ],
    )
    def k(a_hbm, src_hbm, dst_hbm, out_hbm, sidx_v, didx_v, rows_v, zb_v, acc, sem):
        cid = lax.axis_index("c")
        sid = lax.axis_index("s")
        w = cid * NS + sid
        pltpu.sync_copy(src_hbm.at[w], sidx_v)
        pltpu.sync_copy(dst_hbm.at[w], didx_v)

        def fillz(i, carry):
            for t in range(d // LANES):
                zb_v[i, pl.ds(t * LANES, LANES)] = jnp.zeros((LANES,), _f32)
            return carry

        lax.fori_loop(0, CH, fillz, 0)
        base = sid * rpt
        for z in range(nz):
            pltpu.sync_copy(zb_v, acc.at[pl.ds(base + z * CH, CH)])
        plsc.subcore_barrier()

        def body(j, carry):
            pltpu.async_copy(a_hbm.at[sidx_v.at[j]], rows_v, sem).wait()
            pltpu.sync_copy(rows_v, acc.at[didx_v.at[j]], add=True)
            return carry

        lax.fori_loop(0, nch, body, 0)
        plsc.subcore_barrier()
        for z in range(nz):
            sl = pl.ds(base + z * CH, CH)
            pltpu.sync_copy(acc.at[sl], out_hbm.at[cid, sl])

    return k(a, src_r, dst_r)


def _gat_s_chunk(alsg_v, aldg_v, sw_v, groups, head_list, rep):
    for g in range(groups):
        rid = g * LANES + lax.iota(_i32, LANES)
        for li, hh in enumerate(head_list):
            hcol = jnp.full((LANES,), hh, _i32)
            e = (plsc.load_gather(alsg_v, [rid, hcol]) +
                 plsc.load_gather(aldg_v, [rid, hcol]))
            e = jnp.where(e < 0.0, e * 0.2, e)
            s = jnp.exp(e)
            for r in range(rep):
                plsc.store_scatter(
                    sw_v, [rid, jnp.full((LANES,), li + 4 * r, _i32)], s)


def _sc_gat_den(als16, ald16, src_r, dst_r, n):
    heads = 4
    nw, nch, c = src_r.shape
    groups = c // LANES
    nz, rpt, npad = _padrows(n)

    @functools.partial(
        pl.kernel,
        out_type=jax.ShapeDtypeStruct((NC, npad, 16), _f32),
        mesh=_mesh(),
        compiler_params=pltpu.CompilerParams(use_tc_tiling_on_sc=False, needs_layout_passes=False),
        scratch_types=[
            pltpu.VMEM((nch, c), _i32),
            pltpu.VMEM((nch, c), _i32),
            pltpu.VMEM((c, 16), _f32),
            pltpu.VMEM((c, 16), _f32),
            pltpu.VMEM((c, 16), _f32),
            pltpu.VMEM((CH, 16), _f32),
            pltpu.VMEM_SHARED((npad, 16), _f32),
            pltpu.SemaphoreType.DMA,
        ],
    )
    def k(als_hbm, ald_hbm, src_hbm, dst_hbm, den_hbm,
          sidx_v, didx_v, alsg_v, aldg_v, sw_v, zd_v, dacc, sem):
        cid = lax.axis_index("c")
        sid = lax.axis_index("s")
        w = cid * NS + sid
        pltpu.sync_copy(src_hbm.at[w], sidx_v)
        pltpu.sync_copy(dst_hbm.at[w], didx_v)

        def fillz(i, carry):
            zd_v[i, :] = jnp.zeros((LANES,), _f32)
            return carry

        lax.fori_loop(0, CH, fillz, 0)
        base = sid * rpt
        for z in range(nz):
            pltpu.sync_copy(zd_v, dacc.at[pl.ds(base + z * CH, CH)])
        plsc.subcore_barrier()

        def body(j, carry):
            cpa = pltpu.async_copy(als_hbm.at[sidx_v.at[j]], alsg_v, sem)
            cpb = pltpu.async_copy(ald_hbm.at[didx_v.at[j]], aldg_v, sem)
            cpa.wait()
            cpb.wait()
            _gat_s_chunk(alsg_v, aldg_v, sw_v, groups, list(range(heads)), 4)
            pltpu.sync_copy(sw_v, dacc.at[didx_v.at[j]], add=True)
            return carry

        lax.fori_loop(0, nch, body, 0)
        plsc.subcore_barrier()
        for z in range(nz):
            sl = pl.ds(base + z * CH, CH)
            pltpu.sync_copy(dacc.at[sl], den_hbm.at[cid, sl])

    return k(als16, ald16, src_r, dst_r)


def _sc_gat_num(h_slab, als16, ald16, src_r, dst_r, n, h0, hps):
    d = h_slab.shape[1]
    nw, nch, c = src_r.shape
    groups = c // LANES
    nz, rpt, npad = _padrows(n)

    @functools.partial(
        pl.kernel,
        out_type=jax.ShapeDtypeStruct((NC, npad, d), _f32),
        mesh=_mesh(),
        compiler_params=pltpu.CompilerParams(use_tc_tiling_on_sc=False, needs_layout_passes=False),
        scratch_types=[
            pltpu.VMEM((nch, c), _i32),
            pltpu.VMEM((nch, c), _i32),
            pltpu.VMEM((c, d), _f32),
            pltpu.VMEM((c, 16), _f32),
            pltpu.VMEM((c, 16), _f32),
            pltpu.VMEM((c, 16), _f32),
            pltpu.VMEM_SHARED((npad, d), _f32),
            pltpu.SemaphoreType.DMA,
        ],
    )
    def k(h_hbm, als_hbm, ald_hbm, src_hbm, dst_hbm, num_hbm,
          sidx_v, didx_v, rows_v, alsg_v, aldg_v, sw_v, nacc, sem):
        cid = lax.axis_index("c")
        sid = lax.axis_index("s")
        w = cid * NS + sid
        pltpu.sync_copy(src_hbm.at[w], sidx_v)
        pltpu.sync_copy(dst_hbm.at[w], didx_v)

        def fillz(i, carry):
            for t in range(d // LANES):
                rows_v[i, pl.ds(t * LANES, LANES)] = jnp.zeros((LANES,), _f32)
            return carry

        lax.fori_loop(0, c, fillz, 0)
        base = sid * rpt
        for z in range(nz):
            pltpu.sync_copy(rows_v, nacc.at[pl.ds(base + z * CH, CH)])
        plsc.subcore_barrier()

        def body(j, carry):
            cph = pltpu.async_copy(h_hbm.at[sidx_v.at[j]], rows_v, sem)
            cpa = pltpu.async_copy(als_hbm.at[sidx_v.at[j]], alsg_v, sem)
            cpb = pltpu.async_copy(ald_hbm.at[didx_v.at[j]], aldg_v, sem)
            cph.wait()
            cpa.wait()
            cpb.wait()
            _gat_s_chunk(alsg_v, aldg_v, sw_v, groups,
                         list(range(h0, h0 + hps)), 1)

            def scale(i, carry2):
                for li in range(hps):
                    sp = plsc.load_gather(
                        sw_v,
                        [jnp.full((LANES,), i, _i32), jnp.full((LANES,), li, _i32)])
                    for t in range(2):
                        sl = pl.ds(li * 32 + t * LANES, LANES)
                        rows_v[i, sl] = rows_v[i, sl] * sp
                return carry2

            lax.fori_loop(0, c, scale, 0)
            pltpu.sync_copy(rows_v, nacc.at[didx_v.at[j]], add=True)
            return carry

        lax.fori_loop(0, nch, body, 0)
        plsc.subcore_barrier()
        for z in range(nz):
            sl = pl.ds(base + z * CH, CH)
            pltpu.sync_copy(nacc.at[sl], num_hbm.at[cid, sl])

    return k(h_slab, als16, ald16, src_r, dst_r)


def _tc1(x, w1, degp):
    n, din = x.shape
    dout = w1.shape[1]
    half = dout // 2

    def body(x_ref, w_ref, degp_ref, alo_ref, ahi_ref, dinv_ref):
        dsum = degp_ref[0] + degp_ref[1]
        deg = dsum[0:n, 0:1] + 1.0
        dinv = lax.rsqrt(deg)
        a1 = jnp.dot(x_ref[...], w_ref[...], preferred_element_type=_f32) * dinv
        alo_ref[...] = a1[:, 0:half]
        ahi_ref[...] = a1[:, half:dout]
        dinv_ref[...] = dinv

    return pl.pallas_call(
        body,
        out_shape=(jax.ShapeDtypeStruct((n, half), _f32),
                   jax.ShapeDtypeStruct((n, half), _f32),
                   jax.ShapeDtypeStruct((n, 1), _f32)),
    )(x, w1, degp)


def _tc_gcn_next(s_parts, a_parts, dinv, b, g, be, w_next):
    n = dinv.shape[0]
    dout = w_next.shape[1]
    ns, na = len(s_parts), len(a_parts)

    def body(*refs):
        s_refs = refs[:ns]
        a_refs = refs[ns:ns + na]
        dinv_ref, b_ref, g_ref, be_ref, w_ref = refs[ns + na:ns + na + 5]
        out_ref = refs[-1]
        ssum = jnp.concatenate([(r[0] + r[1])[0:n] for r in s_refs], axis=1)
        a = jnp.concatenate([r[...] for r in a_refs], axis=1)
        y = (ssum + a) * dinv_ref[...] + b_ref[...]
        m = jnp.mean(y, axis=0, keepdims=True)
        v = jnp.mean((y - m) * (y - m), axis=0, keepdims=True)
        hn = g_ref[...] * (y - m) * lax.rsqrt(v + 1e-5) + be_ref[...]
        h = jnp.maximum(hn, 0.0)
        out_ref[...] = jnp.dot(h, w_ref[...],
                               preferred_element_type=_f32) * dinv_ref[...]

    return pl.pallas_call(
        body, out_shape=jax.ShapeDtypeStruct((n, dout), _f32),
    )(*s_parts, *a_parts, dinv, b, g, be, w_next)


def _tc4(s_parts, a, dinv, b, g, be, wg, a_s_mat, a_d_mat):
    n, d = a.shape
    dg = wg.shape[1]
    half = dg // 2
    heads = a_s_mat.shape[1]
    ns = len(s_parts)

    def body(*refs):
        s_refs = refs[:ns]
        (a_ref, dinv_ref, b_ref, g_ref, be_ref, wg_ref, as_ref, ad_ref,
         hlo_ref, hhi_ref, als_ref, ald_ref) = refs[ns:]
        ssum = jnp.concatenate([(r[0] + r[1])[0:n] for r in s_refs], axis=1)
        y = (ssum + a_ref[...]) * dinv_ref[...] + b_ref[...]
        m = jnp.mean(y, axis=0, keepdims=True)
        v = jnp.mean((y - m) * (y - m), axis=0, keepdims=True)
        hn = g_ref[...] * (y - m) * lax.rsqrt(v + 1e-5) + be_ref[...]
        h3 = jnp.maximum(hn, 0.0)
        hh = jnp.dot(h3, wg_ref[...], preferred_element_type=_f32)
        hlo_ref[...] = hh[:, 0:half]
        hhi_ref[...] = hh[:, half:dg]
        als_ref[...] = jnp.dot(hh, as_ref[...], preferred_element_type=_f32)
        ald_ref[...] = jnp.dot(hh, ad_ref[...], preferred_element_type=_f32)

    return pl.pallas_call(
        body,
        out_shape=(jax.ShapeDtypeStruct((n, half), _f32),
                   jax.ShapeDtypeStruct((n, half), _f32),
                   jax.ShapeDtypeStruct((n, heads), _f32),
                   jax.ShapeDtypeStruct((n, heads), _f32)),
    )(*s_parts, a, dinv, b, g, be, wg, a_s_mat, a_d_mat)


def _tc5(num_parts, denp, h_parts, als, ald, bg, rmat, pmat, wc1, bc1, wc2, bc2):
    n = als.shape[0]
    nn, nh = len(num_parts), len(h_parts)

    def body(*refs):
        np_refs = refs[:nn]
        dp_ref = refs[nn]
        h_refs = refs[nn + 1:nn + 1 + nh]
        (als_ref, ald_ref, bg_ref, r_ref, p_ref, wc1_ref, bc1_ref, wc2_ref,
         bc2_ref, out_ref) = refs[nn + 1 + nh:]
        e = als_ref[...][:, 0:4] + ald_ref[...][:, 0:4]
        sself = jnp.exp(jnp.where(e < 0.0, 0.2 * e, e))
        scale = jnp.dot(sself, r_ref[...], preferred_element_type=_f32)
        nsum = jnp.concatenate([(r[0] + r[1])[0:n] for r in np_refs], axis=1)
        hfull = jnp.concatenate([r[...] for r in h_refs], axis=1)
        num = nsum + hfull * scale
        dsum = (dp_ref[0] + dp_ref[1])[0:n]
        den4 = dsum[:, 0:4] + sself
        den = jnp.dot(den4, r_ref[...], preferred_element_type=_f32) + 1e-16
        outh = num / den
        h4 = jnp.maximum(
            jnp.dot(outh, p_ref[...], preferred_element_type=_f32) + bg_ref[...],
            0.0)
        gm = jnp.mean(h4, axis=0, keepdims=True)
        gc = jnp.maximum(
            jnp.dot(gm, wc1_ref[...], preferred_element_type=_f32) + bc1_ref[...],
            0.0)
        out_ref[...] = jnp.dot(gc, wc2_ref[...],
                               preferred_element_type=_f32) + bc2_ref[...]

    return pl.pallas_call(
        body, out_shape=jax.ShapeDtypeStruct((1, wc2.shape[1]), _f32),
        compiler_params=pltpu.CompilerParams(vmem_limit_bytes=100 * 1024 * 1024),
    )(*num_parts, denp, *h_parts, als, ald, bg, rmat, pmat, wc1, bc1, wc2, bc2)


def kernel(x, edge_index, params):
    n, din = x.shape
    e = edge_index.shape[1]
    nch = e // (NW * CH)
    assert e == NW * nch * CH, (e, NW, nch, CH)
    p = params

    src_r = edge_index[0].reshape(NW, nch, CH)
    dst_r = edge_index[1].reshape(NW, nch, CH)

    row = lambda v: v.reshape(1, -1)
    heads, hc = p['ag_s'].shape
    eye_h = jnp.eye(heads, dtype=_f32)
    pad = jnp.zeros((heads * hc, 16 - heads), _f32)
    a_s_mat = jnp.concatenate(
        [(eye_h[:, None, :] * p['ag_s'][:, :, None]).reshape(heads * hc, heads),
         pad], axis=1)
    a_d_mat = jnp.concatenate(
        [(eye_h[:, None, :] * p['ag_d'][:, :, None]).reshape(heads * hc, heads),
         pad], axis=1)
    rmat = (eye_h[:, :, None] * jnp.ones((1, 1, hc), _f32)).reshape(heads, heads * hc)
    pmat = (jnp.ones((heads, 1, 1), _f32) *
            jnp.eye(hc, dtype=_f32)[None] * (1.0 / heads)).reshape(heads * hc, hc)

    degp = _sc_degree(dst_r, n)
    a1_lo, a1_hi, dinv = _tc1(x, p['W1'], degp)
    s1a = _sc_scatter(a1_lo, src_r, dst_r, n)
    s1b = _sc_scatter(a1_hi, src_r, dst_r, n)
    a2 = _tc_gcn_next([s1a, s1b], [a1_lo, a1_hi], dinv,
                      row(p['b1']), row(p['g1']), row(p['be1']), p['W2'])
    s2 = _sc_scatter(a2, src_r, dst_r, n)
    a3 = _tc_gcn_next([s2], [a2], dinv,
                      row(p['b2']), row(p['g2']), row(p['be2']), p['W3'])
    s3 = _sc_scatter(a3, src_r, dst_r, n)
    h_lo, h_hi, als, ald = _tc4([s3], a3, dinv, row(p['b3']), row(p['g3']),
                                row(p['be3']), p['Wg'], a_s_mat, a_d_mat)
    denp = _sc_gat_den(als, ald, src_r, dst_r, n)
    num_lo = _sc_gat_num(h_lo, als, ald, src_r, dst_r, n, 0, 2)
    num_hi = _sc_gat_num(h_hi, als, ald, src_r, dst_r, n, 2, 2)
    return _tc5([num_lo, num_hi], denp, [h_lo, h_hi], als, ald, row(p['bg']),
                rmat, pmat, p['Wc1'], row(p['bc1']), p['Wc2'], row(p['bc2']))
